# baseline (speedup 1.0000x reference)
import jax
import jax.numpy as jnp
from jax import lax
from jax.experimental import pallas as pl
from jax.experimental.pallas import tpu as pltpu
from jax.experimental.pallas import tpu_sc as plsc

N_NODES = 10000
N_EDGES = 320000
D = 128

NC = 2
NS = 16
NW = NC * NS

CHUNK = 128
IDX_BLK = 8
NIDX0 = 20
NIDX1 = 0
BLK_E = IDX_BLK * CHUNK
PADDED_E = NS * (NIDX0 + NIDX1) * BLK_E
E_CORE0 = NS * NIDX0 * BLK_E

ACC_ROWS = 10240
ROWS_PER_TILE = ACC_ROWS // NS
PAD_DST = ACC_ROWS - 1


def _edge_loop(x_hbm, src_hbm, dst_hbm, s, nidx, src_v, dst_v, bufs, sems, acc_sh):
    @pl.loop(0, nidx)
    def _(blk):
        pltpu.sync_copy(src_hbm.at[s].at[blk], src_v)
        pltpu.sync_copy(dst_hbm.at[s].at[blk], dst_v)

        pending = pltpu.async_copy(x_hbm.at[src_v.at[0]], bufs[0], sems[0])
        for j in range(IDX_BLK):
            nxt = None
            if j + 1 < IDX_BLK:
                nxt = pltpu.async_copy(
                    x_hbm.at[src_v.at[j + 1]], bufs[(j + 1) % 2], sems[(j + 1) % 2])
            pending.wait()
            pltpu.sync_copy(bufs[j % 2], acc_sh.at[dst_v.at[j]], add=True)
            if nxt is not None:
                pending = nxt


def _agg_body(x_hbm, srcA_hbm, dstA_hbm, out_hbm,
              src_v, dst_v, rows0, rows1, acc_sh, sem0, sem1):
    c = lax.axis_index("c")
    s = lax.axis_index("s")

    @pl.when(c == 0)
    def _():
        @pl.loop(0, CHUNK)
        def _(i):
            @pl.loop(0, D, step=16)
            def _(j):
                rows0[i, pl.ds(j, 16)] = jnp.zeros((16,), jnp.float32)

        @pl.loop(0, ROWS_PER_TILE // CHUNK)
        def _(k):
            zbase = pl.multiple_of(s * ROWS_PER_TILE + k * CHUNK, 8)
            pltpu.sync_copy(rows0, acc_sh.at[pl.ds(zbase, CHUNK)])

        plsc.subcore_barrier()

        _edge_loop(x_hbm, srcA_hbm, dstA_hbm, s, NIDX0,
                   src_v, dst_v, (rows0, rows1), (sem0, sem1), acc_sh)

        plsc.subcore_barrier()

        obase = pl.multiple_of(s * ROWS_PER_TILE, 8)
        pltpu.sync_copy(acc_sh.at[pl.ds(obase, ROWS_PER_TILE)],
                        out_hbm.at[pl.ds(obase, ROWS_PER_TILE)])


_agg = pl.kernel(
    _agg_body,
    out_type=jax.ShapeDtypeStruct((ACC_ROWS, D), jnp.float32),
    mesh=plsc.VectorSubcoreMesh(core_axis_name="c", subcore_axis_name="s"),
    scratch_types=[
        pltpu.VMEM((IDX_BLK, CHUNK), jnp.int32),
        pltpu.VMEM((IDX_BLK, CHUNK), jnp.int32),
        pltpu.VMEM((CHUNK, D), jnp.float32),
        pltpu.VMEM((CHUNK, D), jnp.float32),
        pltpu.VMEM_SHARED((ACC_ROWS, D), jnp.float32),
        pltpu.SemaphoreType.DMA,
        pltpu.SemaphoreType.DMA,
    ],
)

ROW_BLK = 1000


def _mlp_call(xin, parts, W, b2d, relu):
    def body(x_ref, p_ref, w_ref, b_ref, o_ref):
        a = x_ref[...] + p_ref[...]
        y = jnp.dot(a, w_ref[...], preferred_element_type=jnp.float32)
        y = y + b_ref[...]
        if relu:
            y = jnp.maximum(y, 0.0)
        o_ref[...] = y

    return pl.pallas_call(
        body,
        grid=(N_NODES // ROW_BLK,),
        in_specs=[
            pl.BlockSpec((ROW_BLK, D), lambda i: (i, 0)),
            pl.BlockSpec((ROW_BLK, D), lambda i: (i, 0)),
            pl.BlockSpec((D, D), lambda i: (0, 0)),
            pl.BlockSpec((1, D), lambda i: (0, 0)),
        ],
        out_specs=pl.BlockSpec((ROW_BLK, D), lambda i: (i, 0)),
        out_shape=jax.ShapeDtypeStruct((N_NODES, D), jnp.float32),
    )(xin, parts, W, b2d)


def kernel(x, edge_index, W1, b1, W2, b2):
    pad = PADDED_E - N_EDGES
    src = jnp.concatenate(
        [edge_index[0].astype(jnp.int32), jnp.zeros((pad,), jnp.int32)])
    dst = jnp.concatenate(
        [edge_index[1].astype(jnp.int32), jnp.full((pad,), PAD_DST, jnp.int32)])
    srcA = src.reshape(NS, NIDX0, IDX_BLK, CHUNK)
    dstA = dst.reshape(NS, NIDX0, IDX_BLK, CHUNK)
    b1_2d = b1.reshape(1, D)
    b2_2d = b2.reshape(1, D)

    p1 = _agg(x, srcA, dstA)
    h = _mlp_call(x, p1, W1, b1_2d, relu=True)
    p2 = _agg(h, srcA, dstA)
    out = _mlp_call(h, p2, W2, b2_2d, relu=False)
    return out


# speedup vs baseline: 1.0020x; 1.0020x over previous
import jax
import jax.numpy as jnp
from jax import lax
from jax.experimental import pallas as pl
from jax.experimental.pallas import tpu as pltpu
from jax.experimental.pallas import tpu_sc as plsc

N_NODES = 10000
N_EDGES = 320000
D = 128

NC = 2
NS = 16
NW = NC * NS

CHUNK = 128
IDX_BLK = 8
NIDX0 = 20
NIDX1 = 0
BLK_E = IDX_BLK * CHUNK
PADDED_E = NS * (NIDX0 + NIDX1) * BLK_E
E_CORE0 = NS * NIDX0 * BLK_E

ACC_ROWS = 10240
ROWS_PER_TILE = ACC_ROWS // NS
PAD_DST = ACC_ROWS - 1
SPIN_ITERS = 30000


def _edge_loop(x_hbm, src_hbm, dst_hbm, s, nidx, src_v, dst_v, bufs, sems, acc_sh):
    @pl.loop(0, nidx)
    def _(blk):
        pltpu.sync_copy(src_hbm.at[s].at[blk], src_v)
        pltpu.sync_copy(dst_hbm.at[s].at[blk], dst_v)

        pending = pltpu.async_copy(x_hbm.at[src_v.at[0]], bufs[0], sems[0])
        for j in range(IDX_BLK):
            nxt = None
            if j + 1 < IDX_BLK:
                nxt = pltpu.async_copy(
                    x_hbm.at[src_v.at[j + 1]], bufs[(j + 1) % 2], sems[(j + 1) % 2])
            pending.wait()
            pltpu.sync_copy(bufs[j % 2], acc_sh.at[dst_v.at[j]], add=True)
            if nxt is not None:
                pending = nxt


def _agg_body(x_hbm, srcA_hbm, dstA_hbm, out_hbm,
              src_v, dst_v, rows0, rows1, acc_sh, sem0, sem1):
    c = lax.axis_index("c")
    s = lax.axis_index("s")

    @pl.when(c == 0)
    def _():
        @pl.loop(0, CHUNK)
        def _(i):
            @pl.loop(0, D, step=16)
            def _(j):
                rows0[i, pl.ds(j, 16)] = jnp.zeros((16,), jnp.float32)

        @pl.loop(0, ROWS_PER_TILE // CHUNK)
        def _(k):
            zbase = pl.multiple_of(s * ROWS_PER_TILE + k * CHUNK, 8)
            pltpu.sync_copy(rows0, acc_sh.at[pl.ds(zbase, CHUNK)])

        plsc.subcore_barrier()

        _edge_loop(x_hbm, srcA_hbm, dstA_hbm, s, NIDX0,
                   src_v, dst_v, (rows0, rows1), (sem0, sem1), acc_sh)

        plsc.subcore_barrier()

        obase = pl.multiple_of(s * ROWS_PER_TILE, 8)
        pltpu.sync_copy(acc_sh.at[pl.ds(obase, ROWS_PER_TILE)],
                        out_hbm.at[pl.ds(obase, ROWS_PER_TILE)])

    @pl.when(c == 1)
    def _():
        @pl.loop(0, SPIN_ITERS)
        def _(i):
            rows1[0, pl.ds(0, 16)] = rows1[0, pl.ds(0, 16)] + 1.0


_agg = pl.kernel(
    _agg_body,
    out_type=jax.ShapeDtypeStruct((ACC_ROWS, D), jnp.float32),
    mesh=plsc.VectorSubcoreMesh(core_axis_name="c", subcore_axis_name="s"),
    scratch_types=[
        pltpu.VMEM((IDX_BLK, CHUNK), jnp.int32),
        pltpu.VMEM((IDX_BLK, CHUNK), jnp.int32),
        pltpu.VMEM((CHUNK, D), jnp.float32),
        pltpu.VMEM((CHUNK, D), jnp.float32),
        pltpu.VMEM_SHARED((ACC_ROWS, D), jnp.float32),
        pltpu.SemaphoreType.DMA,
        pltpu.SemaphoreType.DMA,
    ],
)

ROW_BLK = 1000


def _mlp_call(xin, parts, W, b2d, relu):
    def body(x_ref, p_ref, w_ref, b_ref, o_ref):
        a = x_ref[...] + p_ref[...]
        y = jnp.dot(a, w_ref[...], preferred_element_type=jnp.float32)
        y = y + b_ref[...]
        if relu:
            y = jnp.maximum(y, 0.0)
        o_ref[...] = y

    return pl.pallas_call(
        body,
        grid=(N_NODES // ROW_BLK,),
        in_specs=[
            pl.BlockSpec((ROW_BLK, D), lambda i: (i, 0)),
            pl.BlockSpec((ROW_BLK, D), lambda i: (i, 0)),
            pl.BlockSpec((D, D), lambda i: (0, 0)),
            pl.BlockSpec((1, D), lambda i: (0, 0)),
        ],
        out_specs=pl.BlockSpec((ROW_BLK, D), lambda i: (i, 0)),
        out_shape=jax.ShapeDtypeStruct((N_NODES, D), jnp.float32),
    )(xin, parts, W, b2d)


def kernel(x, edge_index, W1, b1, W2, b2):
    pad = PADDED_E - N_EDGES
    src = jnp.concatenate(
        [edge_index[0].astype(jnp.int32), jnp.zeros((pad,), jnp.int32)])
    dst = jnp.concatenate(
        [edge_index[1].astype(jnp.int32), jnp.full((pad,), PAD_DST, jnp.int32)])
    srcA = src.reshape(NS, NIDX0, IDX_BLK, CHUNK)
    dstA = dst.reshape(NS, NIDX0, IDX_BLK, CHUNK)
    b1_2d = b1.reshape(1, D)
    b2_2d = b2.reshape(1, D)

    p1 = _agg(x, srcA, dstA)
    h = _mlp_call(x, p1, W1, b1_2d, relu=True)
    p2 = _agg(h, srcA, dstA)
    out = _mlp_call(h, p2, W2, b2_2d, relu=False)
    return out


# speedup vs baseline: 1.5172x; 1.5142x over previous
import jax
import jax.numpy as jnp
from jax import lax
from jax.experimental import pallas as pl
from jax.experimental.pallas import tpu as pltpu
from jax.experimental.pallas import tpu_sc as plsc

N_NODES = 10000
N_EDGES = 320000
D = 128

NC = 2
NS = 16
NW = NC * NS

CHUNK = 128
IDX_BLK = 8
NIDX0 = 19
NIDX1 = 1
BLK_E = IDX_BLK * CHUNK
PADDED_E = NS * (NIDX0 + NIDX1) * BLK_E
E_CORE0 = NS * NIDX0 * BLK_E

ACC_ROWS = 10240
ROWS_PER_TILE = ACC_ROWS // NS
PAD_DST = ACC_ROWS - 1


def _edge_loop(x_hbm, src_hbm, dst_hbm, s, nidx, src_v, dst_v, bufs, sems, acc_sh):
    @pl.loop(0, nidx)
    def _(blk):
        pltpu.sync_copy(src_hbm.at[s].at[blk], src_v)
        pltpu.sync_copy(dst_hbm.at[s].at[blk], dst_v)

        pending = pltpu.async_copy(x_hbm.at[src_v.at[0]], bufs[0], sems[0])
        for j in range(IDX_BLK):
            nxt = None
            if j + 1 < IDX_BLK:
                nxt = pltpu.async_copy(
                    x_hbm.at[src_v.at[j + 1]], bufs[(j + 1) % 2], sems[(j + 1) % 2])
            pending.wait()
            pltpu.sync_copy(bufs[j % 2], acc_sh.at[dst_v.at[j]], add=True)
            if nxt is not None:
                pending = nxt


def _agg_body(x_hbm, srcA_hbm, dstA_hbm, srcB_hbm, dstB_hbm,
              outA_hbm, outB_hbm,
              src_v, dst_v, rows0, rows1, acc_sh, sem0, sem1):
    c = lax.axis_index("c")
    s = lax.axis_index("s")

    @pl.when(c == 0)
    def _():
        @pl.loop(0, CHUNK)
        def _(i):
            @pl.loop(0, D, step=16)
            def _(j):
                rows0[i, pl.ds(j, 16)] = jnp.zeros((16,), jnp.float32)

        @pl.loop(0, ROWS_PER_TILE // CHUNK)
        def _(k):
            zbase = pl.multiple_of(s * ROWS_PER_TILE + k * CHUNK, 8)
            pltpu.sync_copy(rows0, acc_sh.at[pl.ds(zbase, CHUNK)])

        plsc.subcore_barrier()

        _edge_loop(x_hbm, srcA_hbm, dstA_hbm, s, NIDX0,
                   src_v, dst_v, (rows0, rows1), (sem0, sem1), acc_sh)

        plsc.subcore_barrier()

        obase = pl.multiple_of(s * ROWS_PER_TILE, 8)
        pltpu.sync_copy(acc_sh.at[pl.ds(obase, ROWS_PER_TILE)],
                        outA_hbm.at[pl.ds(obase, ROWS_PER_TILE)])

    @pl.when(c == 1)
    def _():
        @pl.loop(0, CHUNK)
        def _(i):
            @pl.loop(0, D, step=16)
            def _(j):
                rows0[i, pl.ds(j, 16)] = jnp.zeros((16,), jnp.float32)

        @pl.loop(0, ROWS_PER_TILE // CHUNK)
        def _(k):
            zbase = pl.multiple_of(s * ROWS_PER_TILE + k * CHUNK, 8)
            pltpu.sync_copy(rows0, acc_sh.at[pl.ds(zbase, CHUNK)])

        plsc.subcore_barrier()

        _edge_loop(x_hbm, srcB_hbm, dstB_hbm, s, NIDX1,
                   src_v, dst_v, (rows0, rows1), (sem0, sem1), acc_sh)

        plsc.subcore_barrier()

        obase = pl.multiple_of(s * ROWS_PER_TILE, 8)
        pltpu.sync_copy(acc_sh.at[pl.ds(obase, ROWS_PER_TILE)],
                        outB_hbm.at[pl.ds(obase, ROWS_PER_TILE)])


_agg = pl.kernel(
    _agg_body,
    out_type=(jax.ShapeDtypeStruct((ACC_ROWS, D), jnp.float32),
              jax.ShapeDtypeStruct((ACC_ROWS, D), jnp.float32)),
    mesh=plsc.VectorSubcoreMesh(core_axis_name="c", subcore_axis_name="s"),
    scratch_types=[
        pltpu.VMEM((IDX_BLK, CHUNK), jnp.int32),
        pltpu.VMEM((IDX_BLK, CHUNK), jnp.int32),
        pltpu.VMEM((CHUNK, D), jnp.float32),
        pltpu.VMEM((CHUNK, D), jnp.float32),
        pltpu.VMEM_SHARED((ACC_ROWS, D), jnp.float32),
        pltpu.SemaphoreType.DMA,
        pltpu.SemaphoreType.DMA,
    ],
)

ROW_BLK = 1000


def _mlp_call(xin, parts, W, b2d, relu):
    def body(x_ref, pa_ref, pb_ref, w_ref, b_ref, o_ref):
        a = x_ref[...] + pa_ref[...] + pb_ref[...]
        y = jnp.dot(a, w_ref[...], preferred_element_type=jnp.float32)
        y = y + b_ref[...]
        if relu:
            y = jnp.maximum(y, 0.0)
        o_ref[...] = y

    pa, pb = parts
    return pl.pallas_call(
        body,
        grid=(N_NODES // ROW_BLK,),
        in_specs=[
            pl.BlockSpec((ROW_BLK, D), lambda i: (i, 0)),
            pl.BlockSpec((ROW_BLK, D), lambda i: (i, 0)),
            pl.BlockSpec((ROW_BLK, D), lambda i: (i, 0)),
            pl.BlockSpec((D, D), lambda i: (0, 0)),
            pl.BlockSpec((1, D), lambda i: (0, 0)),
        ],
        out_specs=pl.BlockSpec((ROW_BLK, D), lambda i: (i, 0)),
        out_shape=jax.ShapeDtypeStruct((N_NODES, D), jnp.float32),
    )(xin, pa, pb, W, b2d)


def kernel(x, edge_index, W1, b1, W2, b2):
    pad = PADDED_E - N_EDGES
    src = jnp.concatenate(
        [edge_index[0].astype(jnp.int32), jnp.zeros((pad,), jnp.int32)])
    dst = jnp.concatenate(
        [edge_index[1].astype(jnp.int32), jnp.full((pad,), PAD_DST, jnp.int32)])
    srcA = src[:E_CORE0].reshape(NS, NIDX0, IDX_BLK, CHUNK)
    dstA = dst[:E_CORE0].reshape(NS, NIDX0, IDX_BLK, CHUNK)
    srcB = src[E_CORE0:].reshape(NS, NIDX1, IDX_BLK, CHUNK)
    dstB = dst[E_CORE0:].reshape(NS, NIDX1, IDX_BLK, CHUNK)
    b1_2d = b1.reshape(1, D)
    b2_2d = b2.reshape(1, D)

    p1 = _agg(x, srcA, dstA, srcB, dstB)
    h = _mlp_call(x, p1, W1, b1_2d, relu=True)
    p2 = _agg(h, srcA, dstA, srcB, dstB)
    out = _mlp_call(h, p2, W2, b2_2d, relu=False)
    return out
